# Initial kernel scaffold; baseline (speedup 1.0000x reference)
import jax
import jax.numpy as jnp
from jax.experimental import pallas as pl


def kernel(wav):
    raise NotImplementedError("write your pallas kernel here")


# speedup vs baseline: 3.1039x; 3.1039x over previous
import functools

import jax
import jax.numpy as jnp
from jax import lax
from jax.experimental import pallas as pl
from jax.experimental.pallas import tpu as pltpu
from jax.experimental.pallas import tpu_sc as plsc

RATIO = 3
NC, NS, L = 2, 16, 16

CHUNK_OUT = 8000
CHUNK_IN = CHUNK_OUT * RATIO


def _make_sc_kernel(B, n_out):
    out_per_w = n_out // NC
    in_per_w = out_per_w * RATIO
    n_chunks = out_per_w // CHUNK_OUT
    mesh = plsc.VectorSubcoreMesh(core_axis_name="c", subcore_axis_name="s")

    @functools.partial(
        pl.kernel,
        mesh=mesh,
        out_type=jax.ShapeDtypeStruct((B * n_out,), jnp.float32),
        scratch_types=[
            pltpu.VMEM((CHUNK_IN,), jnp.float32),
            pltpu.VMEM((CHUNK_OUT,), jnp.float32),
        ],
        compiler_params=pltpu.CompilerParams(needs_layout_passes=False),
    )
    def k(wav_hbm, out_hbm, in_v, out_v):
        row = lax.axis_index("s")
        half = lax.axis_index("c")
        in_base = row * (n_out * RATIO) + half * in_per_w
        out_base = row * n_out + half * out_per_w
        iota3 = lax.iota(jnp.int32, L) * RATIO

        def chunk_body(c, _):
            pltpu.sync_copy(
                wav_hbm.at[pl.ds(in_base + c * CHUNK_IN, CHUNK_IN)],
                in_v)

            def gather_body(i, _):
                idx = iota3 + i * (L * RATIO)
                out_v[pl.ds(i * L, L)] = plsc.load_gather(in_v, [idx])
                return 0

            lax.fori_loop(0, CHUNK_OUT // L, gather_body, 0)
            pltpu.sync_copy(
                out_v,
                out_hbm.at[pl.ds(out_base + c * CHUNK_OUT, CHUNK_OUT)])
            return 0

        lax.fori_loop(0, n_chunks, chunk_body, 0)

    return k


def kernel(wav):
    wav = wav.reshape(wav.shape[0], -1)
    B, n = wav.shape
    n_out = n // RATIO
    flat = _make_sc_kernel(B, n_out)(wav.reshape(-1))
    return flat.reshape(B, n_out)
